# Initial kernel scaffold; baseline (speedup 1.0000x reference)
import jax
import jax.numpy as jnp
from jax.experimental import pallas as pl


def kernel(locations_0, locations_1, locations_2, locations_3, locations_4, box_cls_0, box_cls_1, box_cls_2, box_cls_3, box_cls_4, box_regression_0, box_regression_1, box_regression_2, box_regression_3, box_regression_4, centerness_0, centerness_1, centerness_2, centerness_3, centerness_4, image_sizes):
    raise NotImplementedError("write your pallas kernel here")


# speedup vs baseline: 7.0640x; 7.0640x over previous
import jax
import jax.numpy as jnp
from jax.experimental import pallas as pl

_PRE_NMS_THRESH = 0.05
_PRE_NMS_TOP_N = 1000
_NMS_THRESH = 0.6
_K = 100
_SCORE_THRESH = 0.5
_FINAL_NMS_THRESH = 0.9


def _score_kernel(cls_ref, ctr_ref, sc_ref, cand_ref, cnt_ref):
    t = pl.program_id(1)
    cls = cls_ref[0]
    ctr = ctr_ref[0]
    cls_s = 1.0 / (1.0 + jnp.exp(-cls))
    ctr_s = 1.0 / (1.0 + jnp.exp(-ctr))
    cand = cls_s > _PRE_NMS_THRESH
    sc_ref[0] = cls_s * ctr_s
    cand_ref[0] = cand.astype(jnp.float32)
    part = jnp.sum(cand.astype(jnp.int32)).reshape(1, 1)

    @pl.when(t == 0)
    def _init():
        cnt_ref[0] = part

    @pl.when(t != 0)
    def _acc():
        cnt_ref[0] += part


def _scores_level(cls_t, ctr_t):
    B, HW, C = cls_t.shape
    T = 4200 if HW % 4200 == 0 else HW
    nt = HW // T
    return pl.pallas_call(
        _score_kernel,
        grid=(B, nt),
        in_specs=[
            pl.BlockSpec((1, T, C), lambda b, t: (b, t, 0)),
            pl.BlockSpec((1, T, 1), lambda b, t: (b, t, 0)),
        ],
        out_specs=[
            pl.BlockSpec((1, T, C), lambda b, t: (b, t, 0)),
            pl.BlockSpec((1, T, C), lambda b, t: (b, t, 0)),
            pl.BlockSpec((1, 1, 1), lambda b, t: (b, 0, 0)),
        ],
        out_shape=[
            jax.ShapeDtypeStruct((B, HW, C), jnp.float32),
            jax.ShapeDtypeStruct((B, HW, C), jnp.float32),
            jax.ShapeDtypeStruct((B, 1, 1), jnp.int32),
        ],
    )(cls_t, ctr_t)


def _cumsum_row(x):
    n = x.shape[1]
    d = 1
    while d < n:
        shifted = jnp.concatenate([jnp.zeros((1, d), x.dtype), x[:, : n - d]], axis=1)
        x = x + shifted
        d *= 2
    return x


def _nms_kernel(wh_ref, lx_ref, ly_ref, reg_ref, sc_ref, lb_ref, va_ref,
                box_ref, kept2_ref):
    n = sc_ref.shape[2]
    h_max = wh_ref[0, 0, 0]
    w_max = wh_ref[0, 0, 1]
    lx = lx_ref[0]
    ly = ly_ref[0]
    reg = reg_ref[0]
    x1 = jnp.clip(lx - reg[0:1], 0.0, w_max)
    y1 = jnp.clip(ly - reg[1:2], 0.0, h_max)
    x2 = jnp.clip(lx + reg[2:3], 0.0, w_max)
    y2 = jnp.clip(ly + reg[3:4], 0.0, h_max)
    box_ref[0, 0:1, :] = x1
    box_ref[0, 1:2, :] = y1
    box_ref[0, 2:3, :] = x2
    box_ref[0, 3:4, :] = y2
    areas = (x2 - x1) * (y2 - y1)
    sc = sc_ref[0]
    lb = lb_ref[0].astype(jnp.float32)
    va = va_ref[0]
    idx = jax.lax.broadcasted_iota(jnp.int32, (1, n), 1)
    zf = jnp.zeros((1, n), jnp.float32)
    onef = jnp.ones((1, n), jnp.float32)

    def greedy(valid_f, thresh, use_labels):
        def body(i, kept):
            sel = idx == i
            x1i = jnp.sum(jnp.where(sel, x1, zf))
            y1i = jnp.sum(jnp.where(sel, y1, zf))
            x2i = jnp.sum(jnp.where(sel, x2, zf))
            y2i = jnp.sum(jnp.where(sel, y2, zf))
            ai = jnp.sum(jnp.where(sel, areas, zf))
            vi = jnp.sum(jnp.where(sel, valid_f, zf))
            xx1 = jnp.maximum(x1i, x1)
            yy1 = jnp.maximum(y1i, y1)
            xx2 = jnp.minimum(x2i, x2)
            yy2 = jnp.minimum(y2i, y2)
            w = jnp.maximum(0.0, xx2 - xx1)
            h = jnp.maximum(0.0, yy2 - yy1)
            inter = w * h
            iou = inter / (ai + areas - inter + 1e-9)
            conflict = kept * jnp.where(idx < i, onef, zf) * \
                jnp.where(iou > thresh, onef, zf)
            if use_labels:
                li = jnp.sum(jnp.where(sel, lb, zf))
                conflict = conflict * jnp.where(lb == li, onef, zf)
            sup = jnp.max(conflict)
            return jnp.where(sel, vi * (1.0 - sup), kept)

        return jax.lax.fori_loop(0, n, body, zf)

    kept = greedy(va, _NMS_THRESH, True)
    n1 = jnp.sum(kept)
    rank = _cumsum_row(kept)
    kth = jnp.max(jnp.where((kept > 0.0) & (rank == float(_K)), sc, -jnp.inf))
    gate = jnp.where(n1 > float(_K),
                     jnp.where(sc >= kth, onef, zf), onef)
    m = kept * gate * jnp.where(sc > _SCORE_THRESH, onef, zf)
    kept2 = greedy(m, _FINAL_NMS_THRESH, False)
    kept2_ref[0] = kept2.astype(jnp.int32)


def _run_nms(wh, lx, ly, reg, sc, lb, va):
    B, _, N = sc.shape
    return pl.pallas_call(
        _nms_kernel,
        grid=(B,),
        in_specs=[
            pl.BlockSpec((1, 1, 2), lambda b: (b, 0, 0)),
            pl.BlockSpec((1, 1, N), lambda b: (b, 0, 0)),
            pl.BlockSpec((1, 1, N), lambda b: (b, 0, 0)),
            pl.BlockSpec((1, 4, N), lambda b: (b, 0, 0)),
            pl.BlockSpec((1, 1, N), lambda b: (b, 0, 0)),
            pl.BlockSpec((1, 1, N), lambda b: (b, 0, 0)),
            pl.BlockSpec((1, 1, N), lambda b: (b, 0, 0)),
        ],
        out_specs=[
            pl.BlockSpec((1, 4, N), lambda b: (b, 0, 0)),
            pl.BlockSpec((1, 1, N), lambda b: (b, 0, 0)),
        ],
        out_shape=[
            jax.ShapeDtypeStruct((B, 4, N), jnp.float32),
            jax.ShapeDtypeStruct((B, 1, N), jnp.int32),
        ],
    )(wh, lx, ly, reg, sc, lb, va)


def kernel(locations_0, locations_1, locations_2, locations_3, locations_4,
           box_cls_0, box_cls_1, box_cls_2, box_cls_3, box_cls_4,
           box_regression_0, box_regression_1, box_regression_2,
           box_regression_3, box_regression_4,
           centerness_0, centerness_1, centerness_2, centerness_3,
           centerness_4, image_sizes):
    locs = [locations_0, locations_1, locations_2, locations_3, locations_4]
    clss = [box_cls_0, box_cls_1, box_cls_2, box_cls_3, box_cls_4]
    regs = [box_regression_0, box_regression_1, box_regression_2,
            box_regression_3, box_regression_4]
    ctrs = [centerness_0, centerness_1, centerness_2, centerness_3,
            centerness_4]

    B = clss[0].shape[0]
    P = _PRE_NMS_TOP_N
    sc_l, lb_l, va_l, loc_l, reg_l = [], [], [], [], []
    for lvl in range(5):
        _, C, H, W = clss[lvl].shape
        HW = H * W
        cls_t = jnp.transpose(clss[lvl], (0, 2, 3, 1)).reshape(B, HW, C)
        ctr_t = ctrs[lvl].reshape(B, 1, HW).transpose(0, 2, 1)
        reg_t = jnp.transpose(regs[lvl], (0, 2, 3, 1)).reshape(B, HW, 4)
        sc, cand, cnt = _scores_level(cls_t, ctr_t)
        cnt = cnt.reshape(B, 1)
        sc_flat = sc.reshape(B, HW * C)
        cand_flat = cand.reshape(B, HW * C)
        p = min(P, HW * C)
        _, i_trunc = jax.lax.top_k(
            jnp.where(cand_flat > 0.0, sc_flat, -jnp.inf), p)
        _, i_comp = jax.lax.top_k(cand_flat, p)
        sel = jnp.where(cnt > p, i_trunc, i_comp)
        valid = jnp.arange(p)[None, :] < jnp.minimum(cnt, p)
        loc_idx = sel // C
        cls_idx = sel % C
        sc_sel = jnp.take_along_axis(sc_flat, sel, axis=1)
        loc_sel = locs[lvl][loc_idx]
        reg_sel = jnp.take_along_axis(reg_t, loc_idx[:, :, None], axis=1)
        sc_l.append(sc_sel)
        lb_l.append((cls_idx + 1).astype(jnp.int32))
        va_l.append(valid)
        loc_l.append(loc_sel)
        reg_l.append(reg_sel)

    sc_all = jnp.concatenate(sc_l, axis=1)
    lb_all = jnp.concatenate(lb_l, axis=1)
    va_all = jnp.concatenate(va_l, axis=1)
    loc_all = jnp.concatenate(loc_l, axis=1)
    reg_all = jnp.concatenate(reg_l, axis=1)

    scores = jnp.sqrt(sc_all)
    order = jnp.argsort(jnp.where(va_all, -scores, jnp.inf), axis=1,
                        stable=True)
    scores_s = jnp.take_along_axis(scores, order, axis=1)
    lb_s = jnp.take_along_axis(lb_all, order, axis=1)
    va_s = jnp.take_along_axis(va_all, order, axis=1).astype(jnp.float32)
    loc_s = jnp.take_along_axis(loc_all, order[:, :, None], axis=1)
    reg_s = jnp.take_along_axis(reg_all, order[:, :, None], axis=1)

    N = scores_s.shape[1]
    wh = (image_sizes - 1).astype(jnp.float32).reshape(B, 1, 2)
    lx = loc_s[:, :, 0].reshape(B, 1, N)
    ly = loc_s[:, :, 1].reshape(B, 1, N)
    reg4 = jnp.transpose(reg_s, (0, 2, 1))

    boxes4, kept2 = _run_nms(wh, lx, ly, reg4,
                             scores_s.reshape(B, 1, N),
                             lb_s.reshape(B, 1, N),
                             va_s.reshape(B, 1, N))
    kept2 = kept2.reshape(B, N)

    _, idx2 = jax.lax.top_k(kept2, _K)
    n2 = jnp.minimum(jnp.sum(kept2, axis=1), _K).astype(jnp.int32)
    mk = jnp.arange(_K)[None, :] < n2[:, None]
    boxes_s = jnp.transpose(boxes4, (0, 2, 1))
    boxes_out = jnp.where(mk[:, :, None],
                          jnp.take_along_axis(boxes_s, idx2[:, :, None],
                                              axis=1), 0.0)
    scores_out = jnp.where(mk, jnp.take_along_axis(scores_s, idx2, axis=1),
                           0.0)
    labels_out = jnp.where(mk, jnp.take_along_axis(lb_s, idx2, axis=1), 0)
    return (boxes_out.astype(jnp.float32), scores_out.astype(jnp.float32),
            labels_out.astype(jnp.int32), n2)


# speedup vs baseline: 7.0641x; 1.0000x over previous
import jax
import jax.numpy as jnp
from jax.experimental import pallas as pl
from jax.experimental.pallas import tpu as pltpu

_PRE_NMS_THRESH = 0.05
_PRE_NMS_TOP_N = 1000
_NMS_THRESH = 0.6
_K = 100
_SCORE_THRESH = 0.5
_FINAL_NMS_THRESH = 0.9


def _score_kernel(cls_ref, ctr_ref, sc_ref, cand_ref, cnt_ref):
    t = pl.program_id(1)
    cls = cls_ref[0]
    ctr = ctr_ref[0]
    cls_s = 1.0 / (1.0 + jnp.exp(-cls))
    ctr_s = 1.0 / (1.0 + jnp.exp(-ctr))
    cand = cls_s > _PRE_NMS_THRESH
    sc_ref[0] = cls_s * ctr_s
    cand_ref[0] = cand.astype(jnp.float32)
    part = jnp.sum(cand.astype(jnp.int32)).reshape(1, 1)

    @pl.when(t == 0)
    def _init():
        cnt_ref[0] = part

    @pl.when(t != 0)
    def _acc():
        cnt_ref[0] += part


def _scores_level(cls_t, ctr_t):
    B, HW, C = cls_t.shape
    T = 4200 if HW % 4200 == 0 else HW
    nt = HW // T
    return pl.pallas_call(
        _score_kernel,
        grid=(B, nt),
        in_specs=[
            pl.BlockSpec((1, T, C), lambda b, t: (b, t, 0)),
            pl.BlockSpec((1, T, 1), lambda b, t: (b, t, 0)),
        ],
        out_specs=[
            pl.BlockSpec((1, T, C), lambda b, t: (b, t, 0)),
            pl.BlockSpec((1, T, C), lambda b, t: (b, t, 0)),
            pl.BlockSpec((1, 1, 1), lambda b, t: (b, 0, 0)),
        ],
        out_shape=[
            jax.ShapeDtypeStruct((B, HW, C), jnp.float32),
            jax.ShapeDtypeStruct((B, HW, C), jnp.float32),
            jax.ShapeDtypeStruct((B, 1, 1), jnp.int32),
        ],
        compiler_params=pltpu.CompilerParams(
            dimension_semantics=("parallel", "arbitrary")),
    )(cls_t, ctr_t)


def _cumsum_row(x):
    n = x.shape[1]
    d = 1
    while d < n:
        shifted = jnp.concatenate([jnp.zeros((1, d), x.dtype), x[:, : n - d]], axis=1)
        x = x + shifted
        d *= 2
    return x


def _nms_kernel(wh_ref, lx_ref, ly_ref, reg_ref, sc_ref, lb_ref, va_ref,
                box_ref, kept2_ref):
    n = sc_ref.shape[2]
    h_max = wh_ref[0, 0, 0]
    w_max = wh_ref[0, 0, 1]
    lx = lx_ref[0]
    ly = ly_ref[0]
    reg = reg_ref[0]
    x1 = jnp.clip(lx - reg[0:1], 0.0, w_max)
    y1 = jnp.clip(ly - reg[1:2], 0.0, h_max)
    x2 = jnp.clip(lx + reg[2:3], 0.0, w_max)
    y2 = jnp.clip(ly + reg[3:4], 0.0, h_max)
    box_ref[0, 0:1, :] = x1
    box_ref[0, 1:2, :] = y1
    box_ref[0, 2:3, :] = x2
    box_ref[0, 3:4, :] = y2
    areas = (x2 - x1) * (y2 - y1)
    sc = sc_ref[0]
    lb = lb_ref[0].astype(jnp.float32)
    va = va_ref[0]
    idx = jax.lax.broadcasted_iota(jnp.int32, (1, n), 1)
    zf = jnp.zeros((1, n), jnp.float32)
    onef = jnp.ones((1, n), jnp.float32)

    def greedy(valid_f, thresh, use_labels):
        def body(i, kept):
            sel = idx == i
            x1i = jnp.sum(jnp.where(sel, x1, zf))
            y1i = jnp.sum(jnp.where(sel, y1, zf))
            x2i = jnp.sum(jnp.where(sel, x2, zf))
            y2i = jnp.sum(jnp.where(sel, y2, zf))
            ai = jnp.sum(jnp.where(sel, areas, zf))
            vi = jnp.sum(jnp.where(sel, valid_f, zf))
            xx1 = jnp.maximum(x1i, x1)
            yy1 = jnp.maximum(y1i, y1)
            xx2 = jnp.minimum(x2i, x2)
            yy2 = jnp.minimum(y2i, y2)
            w = jnp.maximum(0.0, xx2 - xx1)
            h = jnp.maximum(0.0, yy2 - yy1)
            inter = w * h
            iou = inter / (ai + areas - inter + 1e-9)
            conflict = kept * jnp.where(idx < i, onef, zf) * \
                jnp.where(iou > thresh, onef, zf)
            if use_labels:
                li = jnp.sum(jnp.where(sel, lb, zf))
                conflict = conflict * jnp.where(lb == li, onef, zf)
            sup = jnp.max(conflict)
            return jnp.where(sel, vi * (1.0 - sup), kept)

        return jax.lax.fori_loop(0, n, body, zf)

    kept = greedy(va, _NMS_THRESH, True)
    n1 = jnp.sum(kept)
    rank = _cumsum_row(kept)
    kth = jnp.max(jnp.where((kept > 0.0) & (rank == float(_K)), sc, -jnp.inf))
    gate = jnp.where(n1 > float(_K),
                     jnp.where(sc >= kth, onef, zf), onef)
    m = kept * gate * jnp.where(sc > _SCORE_THRESH, onef, zf)
    kept2 = greedy(m, _FINAL_NMS_THRESH, False)
    kept2_ref[0] = kept2.astype(jnp.int32)


def _run_nms(wh, lx, ly, reg, sc, lb, va):
    B, _, N = sc.shape
    return pl.pallas_call(
        _nms_kernel,
        grid=(B,),
        in_specs=[
            pl.BlockSpec((1, 1, 2), lambda b: (b, 0, 0)),
            pl.BlockSpec((1, 1, N), lambda b: (b, 0, 0)),
            pl.BlockSpec((1, 1, N), lambda b: (b, 0, 0)),
            pl.BlockSpec((1, 4, N), lambda b: (b, 0, 0)),
            pl.BlockSpec((1, 1, N), lambda b: (b, 0, 0)),
            pl.BlockSpec((1, 1, N), lambda b: (b, 0, 0)),
            pl.BlockSpec((1, 1, N), lambda b: (b, 0, 0)),
        ],
        out_specs=[
            pl.BlockSpec((1, 4, N), lambda b: (b, 0, 0)),
            pl.BlockSpec((1, 1, N), lambda b: (b, 0, 0)),
        ],
        out_shape=[
            jax.ShapeDtypeStruct((B, 4, N), jnp.float32),
            jax.ShapeDtypeStruct((B, 1, N), jnp.int32),
        ],
        compiler_params=pltpu.CompilerParams(
            dimension_semantics=("parallel",)),
    )(wh, lx, ly, reg, sc, lb, va)


def kernel(locations_0, locations_1, locations_2, locations_3, locations_4,
           box_cls_0, box_cls_1, box_cls_2, box_cls_3, box_cls_4,
           box_regression_0, box_regression_1, box_regression_2,
           box_regression_3, box_regression_4,
           centerness_0, centerness_1, centerness_2, centerness_3,
           centerness_4, image_sizes):
    locs = [locations_0, locations_1, locations_2, locations_3, locations_4]
    clss = [box_cls_0, box_cls_1, box_cls_2, box_cls_3, box_cls_4]
    regs = [box_regression_0, box_regression_1, box_regression_2,
            box_regression_3, box_regression_4]
    ctrs = [centerness_0, centerness_1, centerness_2, centerness_3,
            centerness_4]

    B = clss[0].shape[0]
    P = _PRE_NMS_TOP_N
    sc_l, lb_l, va_l, loc_l, reg_l = [], [], [], [], []
    for lvl in range(5):
        _, C, H, W = clss[lvl].shape
        HW = H * W
        cls_t = jnp.transpose(clss[lvl], (0, 2, 3, 1)).reshape(B, HW, C)
        ctr_t = ctrs[lvl].reshape(B, 1, HW).transpose(0, 2, 1)
        reg_t = jnp.transpose(regs[lvl], (0, 2, 3, 1)).reshape(B, HW, 4)
        sc, cand, cnt = _scores_level(cls_t, ctr_t)
        cnt = cnt.reshape(B, 1)
        sc_flat = sc.reshape(B, HW * C)
        cand_flat = cand.reshape(B, HW * C)
        p = min(P, HW * C)
        _, i_trunc = jax.lax.top_k(
            jnp.where(cand_flat > 0.0, sc_flat, -jnp.inf), p)
        _, i_comp = jax.lax.top_k(cand_flat, p)
        sel = jnp.where(cnt > p, i_trunc, i_comp)
        valid = jnp.arange(p)[None, :] < jnp.minimum(cnt, p)
        loc_idx = sel // C
        cls_idx = sel % C
        sc_sel = jnp.take_along_axis(sc_flat, sel, axis=1)
        loc_sel = locs[lvl][loc_idx]
        reg_sel = jnp.take_along_axis(reg_t, loc_idx[:, :, None], axis=1)
        sc_l.append(sc_sel)
        lb_l.append((cls_idx + 1).astype(jnp.int32))
        va_l.append(valid)
        loc_l.append(loc_sel)
        reg_l.append(reg_sel)

    sc_all = jnp.concatenate(sc_l, axis=1)
    lb_all = jnp.concatenate(lb_l, axis=1)
    va_all = jnp.concatenate(va_l, axis=1)
    loc_all = jnp.concatenate(loc_l, axis=1)
    reg_all = jnp.concatenate(reg_l, axis=1)

    scores = jnp.sqrt(sc_all)
    order = jnp.argsort(jnp.where(va_all, -scores, jnp.inf), axis=1,
                        stable=True)
    scores_s = jnp.take_along_axis(scores, order, axis=1)
    lb_s = jnp.take_along_axis(lb_all, order, axis=1)
    va_s = jnp.take_along_axis(va_all, order, axis=1).astype(jnp.float32)
    loc_s = jnp.take_along_axis(loc_all, order[:, :, None], axis=1)
    reg_s = jnp.take_along_axis(reg_all, order[:, :, None], axis=1)

    N = scores_s.shape[1]
    wh = (image_sizes - 1).astype(jnp.float32).reshape(B, 1, 2)
    lx = loc_s[:, :, 0].reshape(B, 1, N)
    ly = loc_s[:, :, 1].reshape(B, 1, N)
    reg4 = jnp.transpose(reg_s, (0, 2, 1))

    boxes4, kept2 = _run_nms(wh, lx, ly, reg4,
                             scores_s.reshape(B, 1, N),
                             lb_s.reshape(B, 1, N),
                             va_s.reshape(B, 1, N))
    kept2 = kept2.reshape(B, N)

    _, idx2 = jax.lax.top_k(kept2, _K)
    n2 = jnp.minimum(jnp.sum(kept2, axis=1), _K).astype(jnp.int32)
    mk = jnp.arange(_K)[None, :] < n2[:, None]
    boxes_s = jnp.transpose(boxes4, (0, 2, 1))
    boxes_out = jnp.where(mk[:, :, None],
                          jnp.take_along_axis(boxes_s, idx2[:, :, None],
                                              axis=1), 0.0)
    scores_out = jnp.where(mk, jnp.take_along_axis(scores_s, idx2, axis=1),
                           0.0)
    labels_out = jnp.where(mk, jnp.take_along_axis(lb_s, idx2, axis=1), 0)
    return (boxes_out.astype(jnp.float32), scores_out.astype(jnp.float32),
            labels_out.astype(jnp.int32), n2)
